# baseline (speedup 1.0000x reference)
import functools

import jax
import jax.numpy as jnp
from jax import lax
from jax.experimental import pallas as pl
from jax.experimental.pallas import tpu as pltpu
from jax.experimental.pallas import tpu_sc as plsc

BATCH = 16384
DIM = 64
NROWS = 1_000_000

_info = plsc.get_sparse_core_info()
_NC = _info.num_cores
_NS = _info.num_subcores
_NW = _NC * _NS
_ROWS_PER_W = DIM // _NW

_mesh = plsc.VectorSubcoreMesh(core_axis_name="c", subcore_axis_name="s")


@functools.partial(
    pl.kernel,
    mesh=_mesh,
    out_type=jax.ShapeDtypeStruct((DIM, BATCH), jnp.float32),
    compiler_params=pltpu.CompilerParams(use_tc_tiling_on_sc=False),
    scratch_types=[
        pltpu.VMEM((BATCH,), jnp.int32),
        pltpu.VMEM((_ROWS_PER_W, BATCH), jnp.float32),
        pltpu.SemaphoreType.DMA,
    ],
)
def _gather_kernel(idx_hbm, table_hbm, out_hbm, idx_v, row_v, sem):
    wid = lax.axis_index("s") * _NC + lax.axis_index("c")
    pltpu.sync_copy(idx_hbm, idx_v)
    copies = []
    for rowi in range(_ROWS_PER_W):
        c = wid * _ROWS_PER_W + rowi
        copies.append(
            pltpu.async_copy(
                table_hbm.at[c].at[idx_v],
                row_v.at[rowi],
                sem,
            )
        )
    for cp in copies:
        cp.wait()
    for rowi in range(_ROWS_PER_W):
        c = wid * _ROWS_PER_W + rowi
        pltpu.sync_copy(row_v.at[rowi], out_hbm.at[c])


def kernel(batchInds, physiologicalProfile):
    outT = _gather_kernel(batchInds, physiologicalProfile.T)
    return outT.T


# speedup vs baseline: 23.8836x; 23.8836x over previous
import functools

import jax
import jax.numpy as jnp
from jax import lax
from jax.experimental import pallas as pl
from jax.experimental.pallas import tpu as pltpu
from jax.experimental.pallas import tpu_sc as plsc

BATCH = 16384
DIM = 64
NROWS = 1_000_000

_info = plsc.get_sparse_core_info()
_NC = _info.num_cores
_NS = _info.num_subcores
_SEG = BATCH // _NS
_ROWS_PER_SC = DIM // _NC

_mesh = plsc.VectorSubcoreMesh(core_axis_name="c", subcore_axis_name="s")


@functools.partial(
    pl.kernel,
    mesh=_mesh,
    out_type=jax.ShapeDtypeStruct((DIM, BATCH), jnp.float32),
    scratch_types=[
        pltpu.VMEM((_SEG,), jnp.int32),
        pltpu.VMEM((_SEG,), jnp.float32),
        pltpu.VMEM_SHARED((NROWS,), jnp.float32),
        pltpu.SemaphoreType.DMA,
    ],
)
def _gather_kernel(idx_hbm, table_hbm, out_hbm, idx_v, val_v, row_sh, sem):
    sc = lax.axis_index("c")
    sid = lax.axis_index("s")
    seg = sid * _SEG
    pltpu.sync_copy(idx_hbm.at[pl.ds(seg, _SEG)], idx_v)
    for p in range(_ROWS_PER_SC):
        c = 2 * p + sc
        t = c // 8 if isinstance(c, int) else lax.div(c, 8)
        s = lax.rem(c, 8)
        @pl.when(sid == 0)
        def _():
            pltpu.async_copy(table_hbm.at[t].at[s], row_sh, sem).wait()

        plsc.subcore_barrier()
        pltpu.async_copy(row_sh.at[idx_v], val_v, sem).wait()
        pltpu.sync_copy(val_v, out_hbm.at[c].at[pl.ds(seg, _SEG)])
        plsc.subcore_barrier()


def kernel(batchInds, physiologicalProfile):
    table3 = physiologicalProfile.T.reshape(8, DIM // 8, NROWS)
    outT = _gather_kernel(batchInds, table3)
    return outT.T


# speedup vs baseline: 25.8361x; 1.0818x over previous
import functools

import jax
import jax.numpy as jnp
from jax import lax
from jax.experimental import pallas as pl
from jax.experimental.pallas import tpu as pltpu
from jax.experimental.pallas import tpu_sc as plsc

BATCH = 16384
DIM = 64
NROWS = 1_000_000

_info = plsc.get_sparse_core_info()
_NC = _info.num_cores
_NS = _info.num_subcores
_SEG = BATCH // _NS
_ROWS_PER_SC = DIM // _NC
_NPAIR = _ROWS_PER_SC // 2

_mesh = plsc.VectorSubcoreMesh(core_axis_name="c", subcore_axis_name="s")


@functools.partial(
    pl.kernel,
    mesh=_mesh,
    out_type=jax.ShapeDtypeStruct((DIM, BATCH), jnp.float32),
    scratch_types=[
        pltpu.VMEM((_SEG,), jnp.int32),
        pltpu.VMEM((_SEG,), jnp.float32),
        pltpu.VMEM_SHARED((NROWS,), jnp.float32),
        pltpu.VMEM_SHARED((NROWS,), jnp.float32),
        pltpu.SemaphoreType.DMA,
        pltpu.SemaphoreType.DMA,
        pltpu.SemaphoreType.DMA,
    ],
)
def _gather_kernel(idx_hbm, table_hbm, out_hbm, idx_v, val_v, row_a, row_b,
                   sem_a, sem_b, sem_g):
    sc = lax.axis_index("c")
    sid = lax.axis_index("s")
    seg = sid * _SEG

    def issue(c, buf, sem):
        pltpu.async_copy(
            table_hbm.at[lax.div(c, 8)].at[lax.rem(c, 8)], buf, sem
        )

    def drain(buf, sem):
        pltpu.make_async_copy(table_hbm.at[0].at[0], buf, sem).wait()

    def gather_row(c, buf):
        pltpu.async_copy(buf.at[idx_v], val_v, sem_g).wait()
        pltpu.sync_copy(val_v, out_hbm.at[c].at[pl.ds(seg, _SEG)])

    pltpu.sync_copy(idx_hbm.at[pl.ds(seg, _SEG)], idx_v)

    @pl.when(sid == 0)
    def _():
        issue(sc, row_a, sem_a)

    def body(q, carry):
        c_a = 4 * q + sc
        c_b = c_a + 2

        @pl.when(sid == 0)
        def _():
            drain(row_a, sem_a)

        plsc.subcore_barrier()

        @pl.when(sid == 0)
        def _():
            issue(c_b, row_b, sem_b)

        gather_row(c_a, row_a)
        plsc.subcore_barrier()

        @pl.when(sid == 0)
        def _():
            drain(row_b, sem_b)

        @pl.when(jnp.logical_and(sid == 0, q < _NPAIR - 1))
        def _():
            issue(c_a + 4, row_a, sem_a)

        plsc.subcore_barrier()
        gather_row(c_b, row_b)
        return carry

    lax.fori_loop(0, _NPAIR, body, 0)


def kernel(batchInds, physiologicalProfile):
    table3 = physiologicalProfile.T.reshape(8, DIM // 8, NROWS)
    outT = _gather_kernel(batchInds, table3)
    return outT.T


# speedup vs baseline: 30.4804x; 1.1798x over previous
import functools

import jax
import jax.numpy as jnp
from jax import lax
from jax.experimental import pallas as pl
from jax.experimental.pallas import tpu as pltpu
from jax.experimental.pallas import tpu_sc as plsc

BATCH = 16384
DIM = 64
NROWS = 1_000_000

_info = plsc.get_sparse_core_info()
_NC = _info.num_cores
_NS = _info.num_subcores
_SEG = BATCH // _NS
_ROWS_PER_SC = DIM // _NC
_NPAIR = _ROWS_PER_SC // 2

_mesh = plsc.VectorSubcoreMesh(core_axis_name="c", subcore_axis_name="s")


@functools.partial(
    pl.kernel,
    mesh=_mesh,
    out_type=jax.ShapeDtypeStruct((DIM, BATCH), jnp.float32),
    scratch_types=[
        pltpu.VMEM((_SEG,), jnp.int32),
        pltpu.VMEM((_SEG,), jnp.float32),
        pltpu.VMEM_SHARED((NROWS,), jnp.float32),
        pltpu.VMEM_SHARED((NROWS,), jnp.float32),
        pltpu.SemaphoreType.DMA,
        pltpu.SemaphoreType.DMA,
        pltpu.SemaphoreType.DMA,
    ],
)
def _gather_kernel(idx_hbm, table_hbm, out_hbm, idx_v, val_v, row_a, row_b,
                   sem_a, sem_b, sem_g):
    sc = lax.axis_index("c")
    sid = lax.axis_index("s")
    seg = sid * _SEG

    def issue(c, buf, sem):
        pltpu.async_copy(
            table_hbm.at[lax.div(c, 8)].at[lax.rem(c, 8)], buf, sem
        )

    def drain(buf, sem):
        pltpu.make_async_copy(table_hbm.at[0].at[0], buf, sem).wait()

    def gather_row(c, buf):
        pltpu.async_copy(buf.at[idx_v], val_v, sem_g).wait()
        pltpu.sync_copy(val_v, out_hbm.at[c].at[pl.ds(seg, _SEG)])

    pltpu.sync_copy(idx_hbm.at[pl.ds(seg, _SEG)], idx_v)

    @pl.when(sid == 0)
    def _():
        issue(sc, row_a, sem_a)

    def body(q, carry):
        c_a = 4 * q + sc
        c_b = c_a + 2

        plsc.subcore_barrier()

        @pl.when(sid == 0)
        def _():
            issue(c_b, row_b, sem_b)
            drain(row_a, sem_a)

        plsc.subcore_barrier()
        gather_row(c_a, row_a)
        plsc.subcore_barrier()

        @pl.when(jnp.logical_and(sid == 0, q < _NPAIR - 1))
        def _():
            issue(c_a + 4, row_a, sem_a)

        @pl.when(sid == 0)
        def _():
            drain(row_b, sem_b)

        plsc.subcore_barrier()
        gather_row(c_b, row_b)
        return carry

    lax.fori_loop(0, _NPAIR, body, 0)


def kernel(batchInds, physiologicalProfile):
    table3 = physiologicalProfile.T.reshape(8, DIM // 8, NROWS)
    outT = _gather_kernel(batchInds, table3)
    return outT.T
